# Initial kernel scaffold; baseline (speedup 1.0000x reference)
import jax
import jax.numpy as jnp
from jax.experimental import pallas as pl


def kernel(x_in, edge_index, edge_atts, params):
    raise NotImplementedError("write your pallas kernel here")


# speedup vs baseline: 1.0801x; 1.0801x over previous
import functools
import math

import jax
import jax.numpy as jnp
from jax.experimental import pallas as pl
from jax.experimental.pallas import tpu as pltpu


def _proj_body(x_ref, w_ref, b_ref, wet_ref, o_ref):
    x = x_ref[...]
    qkvr = jnp.dot(x, w_ref[...], preferred_element_type=jnp.float32) + b_ref[...]
    q = qkvr[:, :128]
    qwe = jnp.dot(q, wet_ref[...], preferred_element_type=jnp.float32)
    o_ref[:, :512] = qkvr
    o_ref[:, 512:] = qwe


def _projections(x, p):
    n, d = x.shape
    w_all = jnp.concatenate([p["Wq"], p["Wk"], p["Wv"], p["Wskip"]], axis=1)
    b_all = jnp.concatenate([p["bq"], p["bk"], p["bv"], p["bskip"]])[None, :]
    wet = jnp.zeros((128, 128), jnp.float32).at[:, :16].set(p["We"].T)
    blk = 1000
    out = pl.pallas_call(
        _proj_body,
        grid=(n // blk,),
        in_specs=[
            pl.BlockSpec((blk, d), lambda i: (i, 0)),
            pl.BlockSpec((d, 512), lambda i: (0, 0)),
            pl.BlockSpec((1, 512), lambda i: (0, 0)),
            pl.BlockSpec((128, 128), lambda i: (0, 0)),
        ],
        out_specs=pl.BlockSpec((blk, 640), lambda i: (i, 0)),
        out_shape=jax.ShapeDtypeStruct((n, 640), jnp.float32),
    )(x, w_all, b_all, wet)
    return (out[:, 0:128], out[:, 128:256], out[:, 256:384], out[:, 384:512],
            out[:, 512:528])


def _combine_body(out_ref, ea_ref, r_ref, we_ref, u_ref, w_ref, o_ref):
    agg = out_ref[...] + jnp.dot(ea_ref[...], we_ref[...],
                                 preferred_element_type=jnp.float32)
    r = r_ref[...]
    b = jnp.sum(agg * u_ref[...] + r * w_ref[...], axis=-1, keepdims=True)
    beta = jax.nn.sigmoid(b)
    y = beta * r + (1.0 - beta) * agg
    o_ref[...] = jnp.where(y > 0, y, 0.01 * y)


def _combine(out_acc, ea_acc, r, p):
    n, c = r.shape
    wb = p["Wbeta"][:, 0]
    u = (wb[:c] + wb[2 * c:])[None, :]
    w = (wb[c:2 * c] - wb[2 * c:])[None, :]
    blk = 1000
    return pl.pallas_call(
        _combine_body,
        grid=(n // blk,),
        in_specs=[
            pl.BlockSpec((blk, c), lambda i: (i, 0)),
            pl.BlockSpec((blk, 16), lambda i: (i, 0)),
            pl.BlockSpec((blk, c), lambda i: (i, 0)),
            pl.BlockSpec((16, c), lambda i: (0, 0)),
            pl.BlockSpec((1, c), lambda i: (0, 0)),
            pl.BlockSpec((1, c), lambda i: (0, 0)),
        ],
        out_specs=pl.BlockSpec((blk, c), lambda i: (i, 0)),
        out_shape=jax.ShapeDtypeStruct((n, c), jnp.float32),
    )(out_acc, ea_acc, r, p["We"], u, w)


def _mlp_body(x1_ref, x2_ref, w1_ref, w2_ref, b_ref, o_ref):
    o_ref[...] = (jnp.dot(x1_ref[...], w1_ref[...], preferred_element_type=jnp.float32)
                  + jnp.dot(x2_ref[...], w2_ref[...], preferred_element_type=jnp.float32)
                  + b_ref[...])


def _mlp(x1, x2, w, b):
    n, c = x1.shape
    nout = w.shape[1]
    w_pad = jnp.zeros((w.shape[0], 128), jnp.float32).at[:, :nout].set(w)
    b_pad = jnp.zeros((1, 128), jnp.float32).at[0, :nout].set(b)
    blk = 1000
    out = pl.pallas_call(
        _mlp_body,
        grid=(n // blk,),
        in_specs=[
            pl.BlockSpec((blk, c), lambda i: (i, 0)),
            pl.BlockSpec((blk, c), lambda i: (i, 0)),
            pl.BlockSpec((c, 128), lambda i: (0, 0)),
            pl.BlockSpec((c, 128), lambda i: (0, 0)),
            pl.BlockSpec((1, 128), lambda i: (0, 0)),
        ],
        out_specs=pl.BlockSpec((blk, 128), lambda i: (i, 0)),
        out_shape=jax.ShapeDtypeStruct((n, 128), jnp.float32),
    )(x1, x2, w_pad[:c], w_pad[c:], b_pad)
    return out[:, :nout]


def _edge_phase(q, k, v, qwe, src, dst, edge_atts, c):
    n = q.shape[0]
    alpha = (jnp.sum(q[dst] * k[src], axis=-1)
             + jnp.sum(edge_atts * qwe[dst], axis=-1)) / math.sqrt(c)
    amax = jax.ops.segment_max(alpha, dst, num_segments=n)
    amax = jnp.where(jnp.isfinite(amax), amax, 0.0)
    ex = jnp.exp(alpha - amax[dst])
    denom = jax.ops.segment_sum(ex, dst, num_segments=n)
    attn = ex / (denom[dst] + 1e-16)
    out = jax.ops.segment_sum(attn[:, None] * v[src], dst, num_segments=n)
    ea = jax.ops.segment_sum(attn[:, None] * edge_atts, dst, num_segments=n)
    return out, ea


def _layer(x, src, dst, edge_atts, p):
    q, k, v, r, qwe = _projections(x, p)
    out, ea = _edge_phase(q, k, v, qwe, src, dst, edge_atts, q.shape[1])
    return _combine(out, ea, r, p)


def kernel(x_in, edge_index, edge_atts, params):
    src, dst = edge_index[0], edge_index[1]
    x1 = _layer(x_in, src, dst, edge_atts, params["l1"])
    x2 = _layer(x1, src, dst, edge_atts, params["l2"])
    return _mlp(x1, x2, params["mlp_W"], params["mlp_b"])


# speedup vs baseline: 3.7620x; 3.4830x over previous
import functools
import math

import jax
import jax.numpy as jnp
from jax import lax
from jax.experimental import pallas as pl
from jax.experimental.pallas import tpu as pltpu
from jax.experimental.pallas import tpu_sc as plsc

NC = 2
NS = 16
NW = NC * NS
L = 16
C = 128
NPAD = 10240
EPAD = 327680
EW = EPAD // NW
CHUNK = 128
NCHUNK = EW // CHUNK
NSLICE = NPAD // NS

_mesh = plsc.VectorSubcoreMesh(core_axis_name="c", subcore_axis_name="s",
                               num_cores=NC, num_subcores=NS)
_sc_params = pltpu.CompilerParams(needs_layout_passes=False,
                                  use_tc_tiling_on_sc=False)


def _wid():
    return lax.axis_index("s") * NC + lax.axis_index("c")


def _proj_body(x_ref, w_ref, b_ref, wet_ref, o_ref):
    x = x_ref[...]
    qkvr = jnp.dot(x, w_ref[...], preferred_element_type=jnp.float32) + b_ref[...]
    q = qkvr[:, :C]
    qwe = jnp.dot(q, wet_ref[...], preferred_element_type=jnp.float32)
    o_ref[:, :4 * C] = qkvr
    o_ref[:, 4 * C:] = qwe


def _projections(x, p):
    n, d = x.shape
    w_all = jnp.concatenate([p["Wq"], p["Wk"], p["Wv"], p["Wskip"]], axis=1)
    b_all = jnp.concatenate([p["bq"], p["bk"], p["bv"], p["bskip"]])[None, :]
    wet = jnp.zeros((C, 128), jnp.float32).at[:, :16].set(p["We"].T)
    blk = 640
    out = pl.pallas_call(
        _proj_body,
        grid=(n // blk,),
        in_specs=[
            pl.BlockSpec((blk, d), lambda i: (i, 0)),
            pl.BlockSpec((d, 4 * C), lambda i: (0, 0)),
            pl.BlockSpec((1, 4 * C), lambda i: (0, 0)),
            pl.BlockSpec((C, 128), lambda i: (0, 0)),
        ],
        out_specs=pl.BlockSpec((blk, 4 * C + 128), lambda i: (i, 0)),
        out_shape=jax.ShapeDtypeStruct((n, 4 * C + 128), jnp.float32),
    )(x, w_all, b_all, wet)
    return (out[:, 0:C], out[:, C:2 * C], out[:, 2 * C:3 * C],
            out[:, 3 * C:4 * C], out[:, 4 * C:4 * C + 16])


def _amax_combine_body(p_ref, o_ref):
    m = jnp.max(p_ref[...], axis=0)
    o_ref[...] = jnp.where(jnp.isfinite(m), m, 0.0)


def _amax_combine(amax_part):
    blk = 2048
    return pl.pallas_call(
        _amax_combine_body,
        grid=(NPAD // blk,),
        in_specs=[pl.BlockSpec((NW, blk), lambda i: (0, i))],
        out_specs=pl.BlockSpec((blk,), lambda i: (i,)),
        out_shape=jax.ShapeDtypeStruct((NPAD,), jnp.float32),
    )(amax_part)


def _denom_combine_body(p_ref, o_ref):
    o_ref[...] = p_ref[0] + p_ref[1] + 1e-16


def _denom_combine(denom_part):
    blk = 2048
    return pl.pallas_call(
        _denom_combine_body,
        grid=(NPAD // blk,),
        in_specs=[pl.BlockSpec((NC, blk), lambda i: (0, i))],
        out_specs=pl.BlockSpec((blk,), lambda i: (i,)),
        out_shape=jax.ShapeDtypeStruct((NPAD,), jnp.float32),
    )(denom_part)


def _combine_body(op_ref, ep_ref, r_ref, we_ref, u_ref, w_ref, o_ref):
    agg = op_ref[0] + op_ref[1] + jnp.dot(
        ep_ref[0] + ep_ref[1], we_ref[...], preferred_element_type=jnp.float32)
    r = r_ref[...]
    b = jnp.sum(agg * u_ref[...] + r * w_ref[...], axis=-1, keepdims=True)
    beta = jax.nn.sigmoid(b)
    y = beta * r + (1.0 - beta) * agg
    o_ref[...] = jnp.where(y > 0, y, 0.01 * y)


def _combine(out_part, ea_part, r, p):
    n, c = r.shape
    wb = p["Wbeta"][:, 0]
    u = (wb[:c] + wb[2 * c:])[None, :]
    w = (wb[c:2 * c] - wb[2 * c:])[None, :]
    blk = 640
    return pl.pallas_call(
        _combine_body,
        grid=(n // blk,),
        in_specs=[
            pl.BlockSpec((NC, blk, c), lambda i: (0, i, 0)),
            pl.BlockSpec((NC, blk, 16), lambda i: (0, i, 0)),
            pl.BlockSpec((blk, c), lambda i: (i, 0)),
            pl.BlockSpec((16, c), lambda i: (0, 0)),
            pl.BlockSpec((1, c), lambda i: (0, 0)),
            pl.BlockSpec((1, c), lambda i: (0, 0)),
        ],
        out_specs=pl.BlockSpec((blk, c), lambda i: (i, 0)),
        out_shape=jax.ShapeDtypeStruct((n, c), jnp.float32),
    )(out_part, ea_part, r, p["We"], u, w)


def _mlp_body(x1_ref, x2_ref, w1_ref, w2_ref, b_ref, o_ref):
    o_ref[...] = (jnp.dot(x1_ref[...], w1_ref[...], preferred_element_type=jnp.float32)
                  + jnp.dot(x2_ref[...], w2_ref[...], preferred_element_type=jnp.float32)
                  + b_ref[...])


def _mlp(x1, x2, w, b):
    n, c = x1.shape
    nout = w.shape[1]
    w_pad = jnp.zeros((w.shape[0], 128), jnp.float32).at[:, :nout].set(w)
    b_pad = jnp.zeros((1, 128), jnp.float32).at[0, :nout].set(b)
    blk = 640
    out = pl.pallas_call(
        _mlp_body,
        grid=(n // blk,),
        in_specs=[
            pl.BlockSpec((blk, c), lambda i: (i, 0)),
            pl.BlockSpec((blk, c), lambda i: (i, 0)),
            pl.BlockSpec((c, 128), lambda i: (0, 0)),
            pl.BlockSpec((c, 128), lambda i: (0, 0)),
            pl.BlockSpec((1, 128), lambda i: (0, 0)),
        ],
        out_specs=pl.BlockSpec((blk, 128), lambda i: (i, 0)),
        out_shape=jax.ShapeDtypeStruct((n, 128), jnp.float32),
    )(x1, x2, w_pad[:c], w_pad[c:], b_pad)
    return out[:, :nout]


def _scatter_max(tbl, idx16, val16):
    def body(_):
        chk = plsc.load_gather(tbl, [idx16])
        nw = jnp.maximum(chk, val16)
        plsc.store_scatter(tbl, [idx16], nw, mask=nw > chk)
        chk2 = plsc.load_gather(tbl, [idx16])
        return jnp.sum((chk2 < val16).astype(jnp.int32))
    lax.while_loop(lambda c: c > 0, body, jnp.int32(1))


def _sc_alpha_body(q_hbm, k_hbm, qwe_hbm, ea_hbm, src_hbm, dst_hbm,
                   alpha_hbm, amaxp_hbm,
                   idx_s, idx_d, qrows, krows, qwerows, earows, abuf,
                   amax_tbl, sem):
    wid = _wid()
    base = wid * EW
    inv_sqrt_c = 1.0 / math.sqrt(C)
    lane = lax.iota(jnp.int32, L)
    neg = jnp.full((L,), -jnp.inf, jnp.float32)

    def init_i(i, _):
        amax_tbl[pl.ds(i * L, L)] = neg
        return 0
    lax.fori_loop(0, NPAD // L, init_i, 0)

    def chunk(ci, _):
        eb = base + ci * CHUNK
        pltpu.sync_copy(src_hbm.at[pl.ds(eb, CHUNK)], idx_s)
        pltpu.sync_copy(dst_hbm.at[pl.ds(eb, CHUNK)], idx_d)
        cp1 = pltpu.async_copy(k_hbm.at[idx_s], krows, sem)
        cp2 = pltpu.async_copy(q_hbm.at[idx_d], qrows, sem)
        cp3 = pltpu.async_copy(qwe_hbm.at[idx_d], qwerows, sem)
        cp4 = pltpu.async_copy(ea_hbm.at[pl.ds(eb, CHUNK)], earows, sem)
        cp1.wait(); cp2.wait(); cp3.wait(); cp4.wait()

        def group(g, _):
            alpha16 = jnp.zeros((L,), jnp.float32)
            for e in range(L):
                row = g * L + e
                acc = qwerows[row, :] * earows[row, :]
                for j in range(C // L):
                    acc = acc + (qrows[row, pl.ds(j * L, L)]
                                 * krows[row, pl.ds(j * L, L)])
                a = jnp.sum(acc) * inv_sqrt_c
                alpha16 = jnp.where(lane == e, a, alpha16)
            abuf[pl.ds(g * L, L)] = alpha16
            d16 = idx_d[pl.ds(g * L, L)]
            _scatter_max(amax_tbl, d16, alpha16)
            return 0
        lax.fori_loop(0, CHUNK // L, group, 0)
        pltpu.sync_copy(abuf, alpha_hbm.at[pl.ds(eb, CHUNK)])
        return 0
    lax.fori_loop(0, NCHUNK, chunk, 0)
    pltpu.sync_copy(amax_tbl, amaxp_hbm.at[wid])


def _sc_alpha(q, k, qwe, ea, src, dst):
    f = pl.kernel(
        _sc_alpha_body,
        out_type=(jax.ShapeDtypeStruct((EPAD,), jnp.float32),
                  jax.ShapeDtypeStruct((NW, NPAD), jnp.float32)),
        mesh=_mesh,
        compiler_params=_sc_params,
        scratch_types=[
            pltpu.VMEM((CHUNK,), jnp.int32),
            pltpu.VMEM((CHUNK,), jnp.int32),
            pltpu.VMEM((CHUNK, C), jnp.float32),
            pltpu.VMEM((CHUNK, C), jnp.float32),
            pltpu.VMEM((CHUNK, 16), jnp.float32),
            pltpu.VMEM((CHUNK, 16), jnp.float32),
            pltpu.VMEM((CHUNK,), jnp.float32),
            pltpu.VMEM((NPAD,), jnp.float32),
            pltpu.SemaphoreType.DMA,
        ],
    )
    return f(q, k, qwe, ea, src, dst)


def _sc_ex_denom_body(alpha_hbm, dst_hbm, amax_hbm, ex_hbm, denp_hbm,
                      idx_d, abuf, exbuf, zbuf, amax_tbl, den_sh, sem):
    cid = lax.axis_index("c")
    sid = lax.axis_index("s")
    wid = _wid()
    base = wid * EW

    pltpu.sync_copy(amax_hbm, amax_tbl)
    zero = jnp.zeros((L,), jnp.float32)

    def zi(i, _):
        zbuf[pl.ds(i * L, L)] = zero
        return 0
    lax.fori_loop(0, NSLICE // L, zi, 0)
    pltpu.sync_copy(zbuf, den_sh.at[pl.ds(sid * NSLICE, NSLICE)])
    plsc.subcore_barrier()

    def chunk(ci, _):
        eb = base + ci * CHUNK
        pltpu.sync_copy(dst_hbm.at[pl.ds(eb, CHUNK)], idx_d)
        pltpu.sync_copy(alpha_hbm.at[pl.ds(eb, CHUNK)], abuf)

        def group(g, _):
            a16 = abuf[pl.ds(g * L, L)]
            d16 = idx_d[pl.ds(g * L, L)]
            am16 = plsc.load_gather(amax_tbl, [d16])
            exbuf[pl.ds(g * L, L)] = jnp.exp(a16 - am16)
            return 0
        lax.fori_loop(0, CHUNK // L, group, 0)
        pltpu.sync_copy(exbuf, ex_hbm.at[pl.ds(eb, CHUNK)])
        pltpu.sync_copy(exbuf, den_sh.at[idx_d], add=True)
        return 0
    lax.fori_loop(0, NCHUNK, chunk, 0)

    plsc.subcore_barrier()
    pltpu.sync_copy(den_sh.at[pl.ds(sid * NSLICE, NSLICE)],
                    denp_hbm.at[cid, pl.ds(sid * NSLICE, NSLICE)])


def _sc_ex_denom(alpha, dst, amax):
    f = pl.kernel(
        _sc_ex_denom_body,
        out_type=(jax.ShapeDtypeStruct((EPAD,), jnp.float32),
                  jax.ShapeDtypeStruct((NC, NPAD), jnp.float32)),
        mesh=_mesh,
        compiler_params=_sc_params,
        scratch_types=[
            pltpu.VMEM((CHUNK,), jnp.int32),
            pltpu.VMEM((CHUNK,), jnp.float32),
            pltpu.VMEM((CHUNK,), jnp.float32),
            pltpu.VMEM((NSLICE,), jnp.float32),
            pltpu.VMEM((NPAD,), jnp.float32),
            pltpu.VMEM_SHARED((NPAD,), jnp.float32),
            pltpu.SemaphoreType.DMA,
        ],
    )
    return f(alpha, dst, amax)


def _sc_aggregate_body(ex_hbm, den_hbm, v_hbm, src_hbm, dst_hbm, ea_hbm,
                       outp_hbm, eap_hbm,
                       idx_s, idx_d, exbuf, vrows, earows, den_tbl,
                       out_sh, ea_sh, sem):
    cid = lax.axis_index("c")
    sid = lax.axis_index("s")
    wid = _wid()
    base = wid * EW

    pltpu.sync_copy(den_hbm, den_tbl)

    zero = jnp.zeros((L,), jnp.float32)

    def zrow(i, _):
        r, c0 = i // (C // L), (i % (C // L)) * L
        vrows[r, pl.ds(c0, L)] = zero
        return 0
    lax.fori_loop(0, CHUNK * C // L, zrow, 0)

    def zrow2(i, _):
        earows[i, :] = zero
        return 0
    lax.fori_loop(0, CHUNK, zrow2, 0)

    for t in range(NSLICE // CHUNK):
        off = sid * NSLICE + t * CHUNK
        pltpu.sync_copy(vrows, out_sh.at[pl.ds(off, CHUNK), :])
        pltpu.sync_copy(earows, ea_sh.at[pl.ds(off, CHUNK), :])
    plsc.subcore_barrier()

    def chunk(ci, _):
        eb = base + ci * CHUNK
        pltpu.sync_copy(src_hbm.at[pl.ds(eb, CHUNK)], idx_s)
        pltpu.sync_copy(dst_hbm.at[pl.ds(eb, CHUNK)], idx_d)
        pltpu.sync_copy(ex_hbm.at[pl.ds(eb, CHUNK)], exbuf)
        cp1 = pltpu.async_copy(v_hbm.at[idx_s], vrows, sem)
        cp2 = pltpu.async_copy(ea_hbm.at[pl.ds(eb, CHUNK)], earows, sem)
        cp1.wait(); cp2.wait()

        def group(g, _):
            ex16 = exbuf[pl.ds(g * L, L)]
            d16 = idx_d[pl.ds(g * L, L)]
            den16 = plsc.load_gather(den_tbl, [d16])
            at16 = ex16 / den16
            for e in range(L):
                row = g * L + e
                s = jnp.broadcast_to(at16[e], (L,))
                for j in range(C // L):
                    vrows[row, pl.ds(j * L, L)] = vrows[row, pl.ds(j * L, L)] * s
                earows[row, :] = earows[row, :] * s
            return 0
        lax.fori_loop(0, CHUNK // L, group, 0)
        pltpu.sync_copy(vrows, out_sh.at[idx_d], add=True)
        pltpu.sync_copy(earows, ea_sh.at[idx_d], add=True)
        return 0
    lax.fori_loop(0, NCHUNK, chunk, 0)

    plsc.subcore_barrier()
    for t in range(NSLICE // CHUNK):
        off = sid * NSLICE + t * CHUNK
        pltpu.sync_copy(out_sh.at[pl.ds(off, CHUNK), :],
                        outp_hbm.at[cid, pl.ds(off, CHUNK), :])
        pltpu.sync_copy(ea_sh.at[pl.ds(off, CHUNK), :],
                        eap_hbm.at[cid, pl.ds(off, CHUNK), :])


def _sc_aggregate(ex, den, v, src, dst, ea):
    f = pl.kernel(
        _sc_aggregate_body,
        out_type=(jax.ShapeDtypeStruct((NC, NPAD, C), jnp.float32),
                  jax.ShapeDtypeStruct((NC, NPAD, 16), jnp.float32)),
        mesh=_mesh,
        compiler_params=_sc_params,
        scratch_types=[
            pltpu.VMEM((CHUNK,), jnp.int32),
            pltpu.VMEM((CHUNK,), jnp.int32),
            pltpu.VMEM((CHUNK,), jnp.float32),
            pltpu.VMEM((CHUNK, C), jnp.float32),
            pltpu.VMEM((CHUNK, 16), jnp.float32),
            pltpu.VMEM((NPAD,), jnp.float32),
            pltpu.VMEM_SHARED((NPAD, C), jnp.float32),
            pltpu.VMEM_SHARED((NPAD, 16), jnp.float32),
            pltpu.SemaphoreType.DMA,
        ],
    )
    return f(ex, den, v, src, dst, ea)


def _layer(x, src, dst, ea, p):
    q, k, v, r, qwe = _projections(x, p)
    alpha, amax_part = _sc_alpha(q, k, qwe, ea, src, dst)
    amax = _amax_combine(amax_part)
    ex, den_part = _sc_ex_denom(alpha, dst, amax)
    den = _denom_combine(den_part)
    out_part, ea_part = _sc_aggregate(ex, den, v, src, dst, ea)
    return _combine(out_part, ea_part, r, p)


def kernel(x_in, edge_index, edge_atts, params):
    n, _ = x_in.shape
    e = edge_index.shape[1]
    x = jnp.zeros((NPAD, x_in.shape[1]), jnp.float32).at[:n].set(x_in)
    pad_idx = jnp.full((EPAD - e,), n, jnp.int32)
    src = jnp.concatenate([edge_index[0], pad_idx])
    dst = jnp.concatenate([edge_index[1], pad_idx])
    ea = jnp.zeros((EPAD, edge_atts.shape[1]), jnp.float32).at[:e].set(edge_atts)

    x1 = _layer(x, src, dst, ea, params["l1"])
    x2 = _layer(x1, src, dst, ea, params["l2"])
    return _mlp(x1, x2, params["mlp_W"], params["mlp_b"])[:n]
